# Initial kernel scaffold; baseline (speedup 1.0000x reference)
import jax
import jax.numpy as jnp
from jax.experimental import pallas as pl


def kernel(x, fourier_bias, key_w, key_b, value_w, value_b, out_w, out_b, gate_w, gate_b, e_w1, e_b1, e_w2, e_b2, ln1_g, ln1_b, ln2_g, ln2_b):
    raise NotImplementedError("write your pallas kernel here")


# speedup vs baseline: 1.2262x; 1.2262x over previous
import jax
import jax.numpy as jnp
from jax.experimental import pallas as pl
from jax.experimental.pallas import tpu as pltpu

_B, _T, _D = 2, 256, 1024
_E = 8
_FF = 4096
_EPS = 1e-5
_NTOK = _B * _T
_BLK = 128
_NBLK = 16
_SLOTS = _NBLK * _BLK
_FBLK = 1024
_NF = _FF // _FBLK


def _fiota(shape, dim):
    return jax.lax.broadcasted_iota(jnp.int32, shape, dim).astype(jnp.float32)


def _ln(h, g, b):
    m = jnp.mean(h, axis=-1, keepdims=True)
    v = jnp.mean((h - m) ** 2, axis=-1, keepdims=True)
    return (h - m) / jnp.sqrt(v + _EPS) * g + b


def _attn_kernel(x_ref, fb_ref, kw_ref, kb_ref, vw_ref, vb_ref, ow_ref, ob_ref,
                 gw_ref, gb_ref, g1_ref, b1_ref,
                 x1_ref, t2e_ref, t2s_ref):
    xb = x_ref[0]
    K = jnp.dot(xb, kw_ref[...], preferred_element_type=jnp.float32) + kb_ref[...]
    V = jnp.dot(xb, vw_ref[...], preferred_element_type=jnp.float32) + vb_ref[...]
    term1 = jnp.sum(K * V, axis=0, keepdims=True)
    term2 = jnp.dot(fb_ref[0], V, preferred_element_type=jnp.float32)
    wv = term2 + term1
    attn = jnp.dot(wv, ow_ref[...], preferred_element_type=jnp.float32) + ob_ref[...]
    x1 = _ln(xb + attn, g1_ref[...], b1_ref[...])
    x1_ref[0] = x1

    logits = jnp.dot(x1, gw_ref[...], preferred_element_type=jnp.float32) + gb_ref[...]
    mx = jnp.max(logits, axis=1, keepdims=True)
    ex = jnp.exp(logits - mx)
    sc = ex / jnp.sum(ex, axis=1, keepdims=True)
    eidx = _fiota((_T, _E), 1)
    m1 = jnp.max(sc, axis=1, keepdims=True)
    e1 = jnp.min(jnp.where(sc == m1, eidx, _E), axis=1, keepdims=True)
    scm = jnp.where(eidx == e1, -jnp.inf, sc)
    m2 = jnp.max(scm, axis=1, keepdims=True)
    e2 = jnp.min(jnp.where(scm == m2, eidx, _E), axis=1, keepdims=True)
    t2e_ref[0] = jnp.concatenate([e1, e2], axis=1)
    t2s_ref[0] = jnp.concatenate([m1, m2], axis=1)


def _route_kernel(t2e_ref, t2s_ref, be_ref, ba_ref, st_ref, sw_ref):
    t2e = t2e_ref[...]
    t2s = t2s_ref[...]
    eidx = _fiota((_NTOK, _E), 1)
    oh1 = (eidx == t2e[:, 0:1]).astype(jnp.float32)
    oh2 = (eidx == t2e[:, 1:2]).astype(jnp.float32)
    mask = oh1 + oh2
    ii = _fiota((_NTOK, _NTOK), 0)
    jj = _fiota((_NTOK, _NTOK), 1)
    ltri = (jj < ii).astype(jnp.float32)
    rank = jnp.dot(ltri, mask, preferred_element_type=jnp.float32)
    counts = jnp.sum(mask, axis=0, keepdims=True)
    nb = jnp.floor((counts + (_BLK - 1)) * (1.0 / _BLK))
    ei = _fiota((_E, _E), 0)
    ej = _fiota((_E, _E), 1)
    utri = (ei <= ej).astype(jnp.float32)
    cum_incl = jnp.dot(nb, utri, preferred_element_type=jnp.float32)
    pad_start = _BLK * (cum_incl - nb)
    total = cum_incl[:, _E - 1:_E]
    slotv = pad_start + rank
    slot1 = jnp.sum(oh1 * slotv, axis=1, keepdims=True)
    slot2 = jnp.sum(oh2 * slotv, axis=1, keepdims=True)
    sidx = _fiota((_NTOK, _SLOTS), 1)
    tcol = _fiota((_NTOK, _SLOTS), 0)
    m1s = slot1 == sidx
    m2s = slot2 == sidx
    st = jnp.sum(jnp.where(m1s, tcol, 0.0) + jnp.where(m2s, tcol, 0.0),
                 axis=0, keepdims=True)
    sw = jnp.sum(jnp.where(m1s, t2s[:, 0:1], 0.0) + jnp.where(m2s, t2s[:, 1:2], 0.0),
                 axis=0, keepdims=True)
    st_ref[...] = st.astype(jnp.int32)
    sw_ref[...] = sw
    kk = _fiota((_NBLK, _E), 0)
    be = jnp.sum((jnp.broadcast_to(cum_incl, (_NBLK, _E)) <= kk).astype(jnp.float32),
                 axis=1, keepdims=True)
    be = jnp.minimum(be, _E - 1)
    be_ref[...] = be.astype(jnp.int32)
    kcol = _fiota((_NBLK, 1), 0)
    ba_ref[...] = (kcol < total).astype(jnp.int32)


def _moe_kernel(be_sm, ba_sm, x1_ref, st_ref, sw_ref, w1_ref, b1_ref,
                w2_ref, b2_ref, g2_ref, b2v_ref, out_ref, xg_ref, acc_ref):
    k = pl.program_id(0)
    f = pl.program_id(1)

    @pl.when((k == 0) & (f == 0))
    def _():
        out_ref[...] = jnp.zeros_like(out_ref)

    @pl.when(ba_sm[k] == 1)
    def _():
        @pl.when(f == 0)
        def _():
            toks = st_ref[0, 0, :]
            gcols = jax.lax.broadcasted_iota(jnp.int32, (_BLK, _NTOK), 1)
            P = (toks[:, None] == gcols).astype(jnp.float32)
            xg_ref[...] = jnp.dot(P, x1_ref[...], preferred_element_type=jnp.float32)
        h = jnp.maximum(
            jnp.dot(xg_ref[...], w1_ref[0], preferred_element_type=jnp.float32)
            + b1_ref[0], 0.0)
        part = jnp.dot(h, w2_ref[0], preferred_element_type=jnp.float32)

        @pl.when(f == 0)
        def _():
            acc_ref[...] = part

        @pl.when(f != 0)
        def _():
            acc_ref[...] = acc_ref[...] + part

        @pl.when(f == _NF - 1)
        def _():
            toks = st_ref[0, 0, :]
            h2 = acc_ref[...] + b2_ref[0]
            contrib = h2 * sw_ref[0, 0, :][:, None]
            srows = jax.lax.broadcasted_iota(jnp.int32, (_NTOK, _BLK), 0)
            Pt = (srows == toks[None, :]).astype(jnp.float32)
            out_ref[...] = out_ref[...] + jnp.dot(
                Pt, contrib, preferred_element_type=jnp.float32)

    @pl.when((k == _NBLK - 1) & (f == _NF - 1))
    def _():
        out_ref[...] = _ln(x1_ref[...] + out_ref[...], g2_ref[...], b2v_ref[...])


def kernel(x, fourier_bias, key_w, key_b, value_w, value_b, out_w, out_b,
           gate_w, gate_b, e_w1, e_b1, e_w2, e_b2, ln1_g, ln1_b, ln2_g, ln2_b):
    f32 = jnp.float32
    row = lambda a: a.reshape(1, -1)

    x1, t2e, t2s = pl.pallas_call(
        _attn_kernel,
        grid=(_B,),
        in_specs=[
            pl.BlockSpec((1, _T, _D), lambda b: (b, 0, 0)),
            pl.BlockSpec((1, _T, _T), lambda b: (b, 0, 0)),
            pl.BlockSpec((_D, _D), lambda b: (0, 0)),
            pl.BlockSpec((1, _D), lambda b: (0, 0)),
            pl.BlockSpec((_D, _D), lambda b: (0, 0)),
            pl.BlockSpec((1, _D), lambda b: (0, 0)),
            pl.BlockSpec((_D, _D), lambda b: (0, 0)),
            pl.BlockSpec((1, _D), lambda b: (0, 0)),
            pl.BlockSpec((_D, _E), lambda b: (0, 0)),
            pl.BlockSpec((1, _E), lambda b: (0, 0)),
            pl.BlockSpec((1, _D), lambda b: (0, 0)),
            pl.BlockSpec((1, _D), lambda b: (0, 0)),
        ],
        out_specs=[
            pl.BlockSpec((1, _T, _D), lambda b: (b, 0, 0)),
            pl.BlockSpec((1, _T, 2), lambda b: (b, 0, 0)),
            pl.BlockSpec((1, _T, 2), lambda b: (b, 0, 0)),
        ],
        out_shape=[
            jax.ShapeDtypeStruct((_B, _T, _D), f32),
            jax.ShapeDtypeStruct((_B, _T, 2), f32),
            jax.ShapeDtypeStruct((_B, _T, 2), f32),
        ],
    )(x, fourier_bias, key_w, row(key_b), value_w, row(value_b),
      out_w, row(out_b), gate_w, row(gate_b), row(ln1_g), row(ln1_b))

    be, ba, st, sw = pl.pallas_call(
        _route_kernel,
        out_shape=[
            jax.ShapeDtypeStruct((_NBLK, 1), jnp.int32),
            jax.ShapeDtypeStruct((_NBLK, 1), jnp.int32),
            jax.ShapeDtypeStruct((1, _SLOTS), jnp.int32),
            jax.ShapeDtypeStruct((1, _SLOTS), f32),
        ],
    )(t2e.reshape(_NTOK, 2), t2s.reshape(_NTOK, 2))

    grid_spec = pltpu.PrefetchScalarGridSpec(
        num_scalar_prefetch=2,
        grid=(_NBLK, _NF),
        in_specs=[
            pl.BlockSpec((_NTOK, _D), lambda k, f, be, ba: (0, 0)),
            pl.BlockSpec((1, 1, _BLK), lambda k, f, be, ba: (k, 0, 0)),
            pl.BlockSpec((1, 1, _BLK), lambda k, f, be, ba: (k, 0, 0)),
            pl.BlockSpec((1, _D, _FBLK),
                         lambda k, f, be, ba: (
                             jnp.where(ba[k] == 1, be[k], _E - 1), 0,
                             jnp.where(ba[k] == 1, f, _NF - 1))),
            pl.BlockSpec((1, 1, _FBLK),
                         lambda k, f, be, ba: (
                             jnp.where(ba[k] == 1, be[k], _E - 1), 0,
                             jnp.where(ba[k] == 1, f, _NF - 1))),
            pl.BlockSpec((1, _FBLK, _D),
                         lambda k, f, be, ba: (
                             jnp.where(ba[k] == 1, be[k], _E - 1),
                             jnp.where(ba[k] == 1, f, _NF - 1), 0)),
            pl.BlockSpec((1, 1, _D),
                         lambda k, f, be, ba: (
                             jnp.where(ba[k] == 1, be[k], _E - 1), 0, 0)),
            pl.BlockSpec((1, _D), lambda k, f, be, ba: (0, 0)),
            pl.BlockSpec((1, _D), lambda k, f, be, ba: (0, 0)),
        ],
        out_specs=pl.BlockSpec((_NTOK, _D), lambda k, f, be, ba: (0, 0)),
        scratch_shapes=[
            pltpu.VMEM((_BLK, _D), f32),
            pltpu.VMEM((_BLK, _D), f32),
        ],
    )
    x2 = pl.pallas_call(
        _moe_kernel,
        grid_spec=grid_spec,
        out_shape=jax.ShapeDtypeStruct((_NTOK, _D), f32),
    )(be.reshape(_NBLK), ba.reshape(_NBLK),
      x1.reshape(_NTOK, _D),
      st.reshape(_NBLK, 1, _BLK), sw.reshape(_NBLK, 1, _BLK),
      e_w1, e_b1.reshape(_E, 1, _FF), e_w2, e_b2.reshape(_E, 1, _D),
      row(ln2_g), row(ln2_b))

    return x2.reshape(_B, _T, _D)


# speedup vs baseline: 1.5980x; 1.3032x over previous
import jax
import jax.numpy as jnp
from jax.experimental import pallas as pl
from jax.experimental.pallas import tpu as pltpu

_B, _T, _D = 2, 256, 1024
_E = 8
_FF = 4096
_EPS = 1e-5
_NTOK = _B * _T
_BLK = 256
_NBLK = 11
_SLOTS = _NBLK * _BLK
_FBLK = 1024
_NF = _FF // _FBLK


def _fiota(shape, dim):
    return jax.lax.broadcasted_iota(jnp.int32, shape, dim).astype(jnp.float32)


def _ln(h, g, b):
    m = jnp.mean(h, axis=-1, keepdims=True)
    v = jnp.mean((h - m) ** 2, axis=-1, keepdims=True)
    return (h - m) / jnp.sqrt(v + _EPS) * g + b


def _attn_route_kernel(x_ref, fb_ref, kw_ref, kb_ref, vw_ref, vb_ref,
                       ow_ref, ob_ref, gw_ref, gb_ref, g1_ref, b1_ref,
                       x1_ref, be_ref, ba_ref, st_ref, sw_ref):
    xb = x_ref[...]
    K = jnp.dot(xb, kw_ref[...], preferred_element_type=jnp.float32) + kb_ref[...]
    V = jnp.dot(xb, vw_ref[...], preferred_element_type=jnp.float32) + vb_ref[...]
    KV = K * V
    wvs = []
    for b in range(_B):
        Vb = V[b * _T:(b + 1) * _T, :]
        t1 = jnp.sum(KV[b * _T:(b + 1) * _T, :], axis=0, keepdims=True)
        t2 = jnp.dot(fb_ref[b], Vb, preferred_element_type=jnp.float32)
        wvs.append(t2 + t1)
    wv = jnp.concatenate(wvs, axis=0)
    attn = jnp.dot(wv, ow_ref[...], preferred_element_type=jnp.float32) + ob_ref[...]
    x1 = _ln(xb + attn, g1_ref[...], b1_ref[...])
    x1_ref[...] = x1

    logits = jnp.dot(x1, gw_ref[...], preferred_element_type=jnp.float32) + gb_ref[...]
    mx = jnp.max(logits, axis=1, keepdims=True)
    ex = jnp.exp(logits - mx)
    sc = ex / jnp.sum(ex, axis=1, keepdims=True)
    eidx = _fiota((_NTOK, _E), 1)
    m1 = jnp.max(sc, axis=1, keepdims=True)
    e1 = jnp.min(jnp.where(sc == m1, eidx, _E), axis=1, keepdims=True)
    scm = jnp.where(eidx == e1, -jnp.inf, sc)
    m2 = jnp.max(scm, axis=1, keepdims=True)
    e2 = jnp.min(jnp.where(scm == m2, eidx, _E), axis=1, keepdims=True)

    oh1 = (eidx == e1).astype(jnp.float32)
    oh2 = (eidx == e2).astype(jnp.float32)
    mask = oh1 + oh2
    ii = _fiota((_NTOK, _NTOK), 0)
    jj = _fiota((_NTOK, _NTOK), 1)
    ltri = (jj < ii).astype(jnp.float32)
    rank = jnp.dot(ltri, mask, preferred_element_type=jnp.float32)
    counts = jnp.sum(mask, axis=0, keepdims=True)
    nb = jnp.floor((counts + (_BLK - 1)) * (1.0 / _BLK))
    ei = _fiota((_E, _E), 0)
    ej = _fiota((_E, _E), 1)
    utri = (ei <= ej).astype(jnp.float32)
    cum_incl = jnp.dot(nb, utri, preferred_element_type=jnp.float32)
    pad_start = _BLK * (cum_incl - nb)
    total = cum_incl[:, _E - 1:_E]
    slotv = pad_start + rank
    slot1 = jnp.sum(oh1 * slotv, axis=1, keepdims=True)
    slot2 = jnp.sum(oh2 * slotv, axis=1, keepdims=True)
    sidx = _fiota((_NTOK, _SLOTS), 1)
    tcol = _fiota((_NTOK, _SLOTS), 0)
    m1s = slot1 == sidx
    m2s = slot2 == sidx
    st = jnp.sum(jnp.where(m1s, tcol, 0.0) + jnp.where(m2s, tcol, 0.0),
                 axis=0, keepdims=True)
    sw = jnp.sum(jnp.where(m1s, m1, 0.0) + jnp.where(m2s, m2, 0.0),
                 axis=0, keepdims=True)
    st_ref[...] = st.astype(jnp.int32)
    sw_ref[...] = sw
    kk = _fiota((_NBLK, _E), 0)
    be = jnp.sum((jnp.broadcast_to(cum_incl, (_NBLK, _E)) <= kk).astype(jnp.float32),
                 axis=1, keepdims=True)
    be = jnp.minimum(be, _E - 1)
    be_ref[...] = be.astype(jnp.int32)
    kcol = _fiota((_NBLK, 1), 0)
    ba_ref[...] = (kcol < total).astype(jnp.int32)


def _moe_kernel(be_sm, ba_sm, x1_ref, st_ref, sw_ref, w1_ref, b1_ref,
                w2_ref, b2_ref, g2_ref, b2v_ref, out_ref, xg_ref, acc_ref):
    k = pl.program_id(0)
    f = pl.program_id(1)

    @pl.when((k == 0) & (f == 0))
    def _():
        out_ref[...] = jnp.zeros_like(out_ref)

    @pl.when(ba_sm[k] == 1)
    def _():
        @pl.when(f == 0)
        def _():
            toks = st_ref[0, 0, :]
            gcols = jax.lax.broadcasted_iota(jnp.int32, (_BLK, _NTOK), 1)
            P = (toks[:, None] == gcols).astype(jnp.float32)
            xg_ref[...] = jnp.dot(P, x1_ref[...], preferred_element_type=jnp.float32)
        h = jnp.maximum(
            jnp.dot(xg_ref[...], w1_ref[0], preferred_element_type=jnp.float32)
            + b1_ref[0], 0.0)
        part = jnp.dot(h, w2_ref[0], preferred_element_type=jnp.float32)

        @pl.when(f == 0)
        def _():
            acc_ref[...] = part

        @pl.when(f != 0)
        def _():
            acc_ref[...] = acc_ref[...] + part

        @pl.when(f == _NF - 1)
        def _():
            toks = st_ref[0, 0, :]
            h2 = acc_ref[...] + b2_ref[0]
            contrib = h2 * sw_ref[0, 0, :][:, None]
            srows = jax.lax.broadcasted_iota(jnp.int32, (_NTOK, _BLK), 0)
            Pt = (srows == toks[None, :]).astype(jnp.float32)
            out_ref[...] = out_ref[...] + jnp.dot(
                Pt, contrib, preferred_element_type=jnp.float32)

    @pl.when((k == _NBLK - 1) & (f == _NF - 1))
    def _():
        out_ref[...] = _ln(x1_ref[...] + out_ref[...], g2_ref[...], b2v_ref[...])


def kernel(x, fourier_bias, key_w, key_b, value_w, value_b, out_w, out_b,
           gate_w, gate_b, e_w1, e_b1, e_w2, e_b2, ln1_g, ln1_b, ln2_g, ln2_b):
    f32 = jnp.float32
    row = lambda a: a.reshape(1, -1)

    x1, be, ba, st, sw = pl.pallas_call(
        _attn_route_kernel,
        out_shape=[
            jax.ShapeDtypeStruct((_NTOK, _D), f32),
            jax.ShapeDtypeStruct((_NBLK, 1), jnp.int32),
            jax.ShapeDtypeStruct((_NBLK, 1), jnp.int32),
            jax.ShapeDtypeStruct((1, _SLOTS), jnp.int32),
            jax.ShapeDtypeStruct((1, _SLOTS), f32),
        ],
    )(x.reshape(_NTOK, _D), fourier_bias, key_w, row(key_b),
      value_w, row(value_b), out_w, row(out_b), gate_w, row(gate_b),
      row(ln1_g), row(ln1_b))

    grid_spec = pltpu.PrefetchScalarGridSpec(
        num_scalar_prefetch=2,
        grid=(_NBLK, _NF),
        in_specs=[
            pl.BlockSpec((_NTOK, _D), lambda k, f, be, ba: (0, 0)),
            pl.BlockSpec((1, 1, _BLK), lambda k, f, be, ba: (k, 0, 0)),
            pl.BlockSpec((1, 1, _BLK), lambda k, f, be, ba: (k, 0, 0)),
            pl.BlockSpec((1, _D, _FBLK),
                         lambda k, f, be, ba: (
                             jnp.where(ba[k] == 1, be[k], _E - 1), 0,
                             jnp.where(ba[k] == 1, f, _NF - 1))),
            pl.BlockSpec((1, 1, _FBLK),
                         lambda k, f, be, ba: (
                             jnp.where(ba[k] == 1, be[k], _E - 1), 0,
                             jnp.where(ba[k] == 1, f, _NF - 1))),
            pl.BlockSpec((1, _FBLK, _D),
                         lambda k, f, be, ba: (
                             jnp.where(ba[k] == 1, be[k], _E - 1),
                             jnp.where(ba[k] == 1, f, _NF - 1), 0)),
            pl.BlockSpec((1, 1, _D),
                         lambda k, f, be, ba: (
                             jnp.where(ba[k] == 1, be[k], _E - 1), 0, 0)),
            pl.BlockSpec((1, _D), lambda k, f, be, ba: (0, 0)),
            pl.BlockSpec((1, _D), lambda k, f, be, ba: (0, 0)),
        ],
        out_specs=pl.BlockSpec((_NTOK, _D), lambda k, f, be, ba: (0, 0)),
        scratch_shapes=[
            pltpu.VMEM((_BLK, _D), f32),
            pltpu.VMEM((_BLK, _D), f32),
        ],
    )
    x2 = pl.pallas_call(
        _moe_kernel,
        grid_spec=grid_spec,
        out_shape=jax.ShapeDtypeStruct((_NTOK, _D), f32),
    )(be.reshape(_NBLK), ba.reshape(_NBLK),
      x1,
      st.reshape(_NBLK, 1, _BLK), sw.reshape(_NBLK, 1, _BLK),
      e_w1, e_b1.reshape(_E, 1, _FF), e_w2, e_b2.reshape(_E, 1, _D),
      row(ln2_g), row(ln2_b))

    return x2.reshape(_B, _T, _D)
